# baseline (speedup 1.0000x reference)
import functools

import jax
import jax.numpy as jnp
from jax import lax
from jax.experimental import pallas as pl
from jax.experimental.pallas import tpu as pltpu
from jax.experimental.pallas import tpu_sc as plsc

BATCH = 4096
SEQ = 200
EMB_DIM = 64
PAD_DIM = 128
VOCAB = 1000000

NC = 2
NS = 16
NW = NC * NS
L = 16

N_BLOCKS = VOCAB // PAD_DIM
BLOCKS_PER_W = N_BLOCKS // NW + 1
BLK_WORDS = PAD_DIM * EMB_DIM

B_TOTAL = BATCH * SEQ
B_PER_W = B_TOTAL // NW
IDX_MINOR = 128
N_STREAMS = B_PER_W // IDX_MINOR
GROUP = 4
ROWS_PER_CHUNK = GROUP * IDX_MINOR
N_CHUNKS = N_STREAMS // GROUP
N_HALF = N_CHUNKS // 2


def _make_relayout():
    mesh = plsc.VectorSubcoreMesh(core_axis_name="c", subcore_axis_name="s")

    @functools.partial(
        pl.kernel,
        out_type=jax.ShapeDtypeStruct((VOCAB * EMB_DIM,), jnp.float32),
        mesh=mesh,
        scratch_types=[
            pltpu.VMEM((EMB_DIM, PAD_DIM), jnp.float32),
            pltpu.VMEM((EMB_DIM, PAD_DIM), jnp.float32),
            pltpu.VMEM((BLK_WORDS,), jnp.float32),
            pltpu.VMEM((BLK_WORDS,), jnp.float32),
            pltpu.SemaphoreType.DMA,
            pltpu.SemaphoreType.DMA,
            pltpu.SemaphoreType.DMA,
            pltpu.SemaphoreType.DMA,
        ],
        compiler_params=pltpu.CompilerParams(
            use_tc_tiling_on_sc=True, needs_layout_passes=False),
    )
    def relayout(tt_hbm, tail_hbm, out_hbm, in_a, in_b, tr_a, tr_b,
                 sem_ia, sem_ib, sem_oa, sem_ob):
        wid = lax.axis_index("s") * NC + lax.axis_index("c")
        iota64 = lax.iota(jnp.int32, L) * EMB_DIM

        def blk(n):
            return n * NW + wid

        def fire_in(it, buf, sem):
            off = pl.multiple_of(it * PAD_DIM, PAD_DIM)
            pltpu.async_copy(
                tt_hbm.at[:, pl.ds(off, PAD_DIM)], buf, sem)

        def wait_in(buf, sem):
            pltpu.make_async_copy(
                tt_hbm.at[:, pl.ds(0, PAD_DIM)], buf, sem).wait()

        def transpose(buf, tr):
            def dq_body(dq, carry):
                for dd in range(4):
                    d = dq * 4 + dd
                    for ib in range(PAD_DIM // L):
                        v = buf[d, pl.ds(ib * L, L)]
                        plsc.store_scatter(
                            tr, [iota64 + (ib * L * EMB_DIM + d)], v)
                return carry
            lax.fori_loop(0, EMB_DIM // 4, dq_body, 0)

        def transpose_tail(buf, tr):
            def dq_body(dq, carry):
                for dd in range(4):
                    d = dq * 4 + dd
                    for ib in range(PAD_DIM // (2 * L)):
                        v = buf[d, pl.ds(ib * L, L)]
                        plsc.store_scatter(
                            tr, [iota64 + (ib * L * EMB_DIM + d)], v)
                return carry
            lax.fori_loop(0, EMB_DIM // 4, dq_body, 0)

        def fire_out(it, tr, sem):
            off = pl.multiple_of(it * BLK_WORDS, BLK_WORDS)
            pltpu.async_copy(
                tr, out_hbm.at[pl.ds(off, BLK_WORDS)], sem)

        def wait_out(tr, sem):
            pltpu.make_async_copy(
                tr, out_hbm.at[pl.ds(0, BLK_WORDS)], sem).wait()

        fire_in(blk(0), in_a, sem_ia)

        def body(k, carry):
            n0 = 2 * k

            @pl.when(blk(n0 + 1) < N_BLOCKS)
            def _():
                fire_in(blk(n0 + 1), in_b, sem_ib)

            @pl.when(blk(n0) < N_BLOCKS)
            def _():
                wait_in(in_a, sem_ia)

                @pl.when(k > 0)
                def _():
                    wait_out(tr_a, sem_oa)
                transpose(in_a, tr_a)
                fire_out(blk(n0), tr_a, sem_oa)

            @pl.when(blk(n0 + 2) < N_BLOCKS)
            def _():
                fire_in(blk(n0 + 2), in_a, sem_ia)

            @pl.when(blk(n0 + 1) < N_BLOCKS)
            def _():
                wait_in(in_b, sem_ib)

                @pl.when(k > 0)
                def _():
                    wait_out(tr_b, sem_ob)
                transpose(in_b, tr_b)
                fire_out(blk(n0 + 1), tr_b, sem_ob)
            return carry

        n_iters = (BLOCKS_PER_W + 1) // 2
        lax.fori_loop(0, n_iters, body, 0)

        wait_out(tr_a, sem_oa)
        wait_out(tr_b, sem_ob)

        @pl.when(wid == 0)
        def _():
            pltpu.sync_copy(tail_hbm, in_a)
            transpose_tail(in_a, tr_a)
            pltpu.sync_copy(
                tr_a.at[pl.ds(0, BLK_WORDS // 2)],
                out_hbm.at[pl.ds(N_BLOCKS * BLK_WORDS, BLK_WORDS // 2)])

    return relayout


def _make_gather():
    mesh = plsc.VectorSubcoreMesh(core_axis_name="c", subcore_axis_name="s")

    @functools.partial(
        pl.kernel,
        out_type=jax.ShapeDtypeStruct((B_TOTAL, PAD_DIM), jnp.float32),
        mesh=mesh,
        scratch_types=[
            pltpu.VMEM((N_STREAMS, IDX_MINOR), jnp.int32),
            pltpu.VMEM((ROWS_PER_CHUNK, EMB_DIM), jnp.float32),
            pltpu.VMEM((ROWS_PER_CHUNK, EMB_DIM), jnp.float32),
            pltpu.SemaphoreType.DMA,
            pltpu.SemaphoreType.DMA,
        ],
        compiler_params=pltpu.CompilerParams(use_tc_tiling_on_sc=False),
    )
    def emb(x_hbm, table_hbm, out_hbm, idx_v, buf_a, buf_b, sem_a, sem_b):
        wid = lax.axis_index("s") * NC + lax.axis_index("c")
        base = wid * B_PER_W
        pltpu.sync_copy(x_hbm.at[wid], idx_v)

        def fire(c, buf, sem):
            for j in range(GROUP):
                pltpu.async_copy(
                    table_hbm.at[idx_v.at[c * GROUP + j]],
                    buf.at[pl.ds(j * IDX_MINOR, IDX_MINOR), :],
                    sem,
                )

        def drain(c, buf, sem):
            for j in range(GROUP):
                pltpu.make_async_copy(
                    table_hbm.at[idx_v.at[c * GROUP + j]],
                    buf.at[pl.ds(j * IDX_MINOR, IDX_MINOR), :],
                    sem,
                ).wait()

        def writeback(c, buf):
            pltpu.sync_copy(
                buf,
                out_hbm.at[pl.ds(base + c * ROWS_PER_CHUNK, ROWS_PER_CHUNK),
                           pl.ds(0, EMB_DIM)],
            )

        fire(0, buf_a, sem_a)

        def body(k, carry):
            c0 = 2 * k
            fire(c0 + 1, buf_b, sem_b)
            drain(c0, buf_a, sem_a)
            writeback(c0, buf_a)

            @pl.when(k < N_HALF - 1)
            def _():
                fire(c0 + 2, buf_a, sem_a)

            drain(c0 + 1, buf_b, sem_b)
            writeback(c0 + 1, buf_b)
            return carry

        lax.fori_loop(0, N_HALF, body, 0)

    return emb


_relayout = _make_relayout()
_emb = _make_gather()


@jax.jit
def kernel(x, table):
    x_r = x.reshape(NW, N_STREAMS, IDX_MINOR).astype(jnp.int32)
    tail_p = jnp.pad(table[N_BLOCKS * PAD_DIM:].T,
                     ((0, 0), (0, PAD_DIM - EMB_DIM)))
    table_lin = _relayout(table.T, tail_p).reshape(VOCAB, EMB_DIM)
    out_p = _emb(x_r, table_lin)
    return out_p[:, :EMB_DIM].reshape(BATCH, SEQ, EMB_DIM)


# speedup vs baseline: 1.2786x; 1.2786x over previous
import functools

import jax
import jax.numpy as jnp
from jax import lax
from jax.experimental import pallas as pl
from jax.experimental.pallas import tpu as pltpu
from jax.experimental.pallas import tpu_sc as plsc

BATCH = 4096
SEQ = 200
EMB_DIM = 64
PAD_DIM = 128
VOCAB = 1000000

NC = 2
NS = 16
NW = NC * NS
L = 16

N_BLOCKS = VOCAB // PAD_DIM
BLOCKS_PER_W = N_BLOCKS // NW + 1
BLK_WORDS = PAD_DIM * EMB_DIM
STRIDE = EMB_DIM + 1

B_TOTAL = BATCH * SEQ
B_PER_W = B_TOTAL // NW
IDX_MINOR = 128
N_STREAMS = B_PER_W // IDX_MINOR
GROUP = 4
ROWS_PER_CHUNK = GROUP * IDX_MINOR
N_CHUNKS = N_STREAMS // GROUP
N_HALF = N_CHUNKS // 2


def _make_relayout():
    mesh = plsc.VectorSubcoreMesh(core_axis_name="c", subcore_axis_name="s")

    @functools.partial(
        pl.kernel,
        out_type=jax.ShapeDtypeStruct((VOCAB * EMB_DIM,), jnp.float32),
        mesh=mesh,
        scratch_types=[
            pltpu.VMEM((EMB_DIM, PAD_DIM), jnp.float32),
            pltpu.VMEM((EMB_DIM, PAD_DIM), jnp.float32),
            pltpu.VMEM((PAD_DIM * STRIDE,), jnp.float32),
            pltpu.VMEM((PAD_DIM * STRIDE,), jnp.float32),
            pltpu.VMEM((BLK_WORDS,), jnp.float32),
            pltpu.VMEM((BLK_WORDS,), jnp.float32),
            pltpu.SemaphoreType.DMA,
            pltpu.SemaphoreType.DMA,
            pltpu.SemaphoreType.DMA,
            pltpu.SemaphoreType.DMA,
        ],
        compiler_params=pltpu.CompilerParams(
            use_tc_tiling_on_sc=True, needs_layout_passes=False),
    )
    def relayout(tt_hbm, tail_hbm, out_hbm, in_a, in_b, st_a, st_b,
                 tr_a, tr_b, sem_ia, sem_ib, sem_oa, sem_ob):
        wid = lax.axis_index("s") * NC + lax.axis_index("c")
        iota65 = lax.iota(jnp.int32, L) * STRIDE

        def blk(n):
            return n * NW + wid

        def fire_in(it, buf, sem):
            off = pl.multiple_of(it * PAD_DIM, PAD_DIM)
            pltpu.async_copy(
                tt_hbm.at[:, pl.ds(off, PAD_DIM)], buf, sem)

        def wait_in(buf, sem):
            pltpu.make_async_copy(
                tt_hbm.at[:, pl.ds(0, PAD_DIM)], buf, sem).wait()

        def transpose(buf, st, n_i=PAD_DIM):
            def dq_body(dq, carry):
                for dd in range(4):
                    d = dq * 4 + dd
                    for ib in range(n_i // L):
                        v = buf[d, pl.ds(ib * L, L)]
                        plsc.store_scatter(
                            st, [iota65 + (ib * L * STRIDE + d)], v)
                return carry
            lax.fori_loop(0, EMB_DIM // 4, dq_body, 0)

        def compact(st, tr, n_i=PAD_DIM):
            def i4_body(i4, carry):
                for di in range(4):
                    i = i4 * 4 + di
                    for k in range(EMB_DIM // L):
                        tr[pl.ds(i * EMB_DIM + k * L, L)] = (
                            st[pl.ds(i * STRIDE + k * L, L)])
                return carry
            lax.fori_loop(0, n_i // 4, i4_body, 0)

        def fire_out(it, tr, sem):
            off = pl.multiple_of(it * BLK_WORDS, BLK_WORDS)
            pltpu.async_copy(
                tr, out_hbm.at[pl.ds(off, BLK_WORDS)], sem)

        def wait_out(tr, sem):
            pltpu.make_async_copy(
                tr, out_hbm.at[pl.ds(0, BLK_WORDS)], sem).wait()

        fire_in(blk(0), in_a, sem_ia)

        def body(k, carry):
            n0 = 2 * k

            @pl.when(blk(n0 + 1) < N_BLOCKS)
            def _():
                fire_in(blk(n0 + 1), in_b, sem_ib)

            @pl.when(blk(n0) < N_BLOCKS)
            def _():
                wait_in(in_a, sem_ia)

                @pl.when(k > 0)
                def _():
                    wait_out(tr_a, sem_oa)
                transpose(in_a, st_a)
                compact(st_a, tr_a)
                fire_out(blk(n0), tr_a, sem_oa)

            @pl.when(blk(n0 + 2) < N_BLOCKS)
            def _():
                fire_in(blk(n0 + 2), in_a, sem_ia)

            @pl.when(blk(n0 + 1) < N_BLOCKS)
            def _():
                wait_in(in_b, sem_ib)

                @pl.when(k > 0)
                def _():
                    wait_out(tr_b, sem_ob)
                transpose(in_b, st_b)
                compact(st_b, tr_b)
                fire_out(blk(n0 + 1), tr_b, sem_ob)
            return carry

        n_iters = (BLOCKS_PER_W + 1) // 2
        lax.fori_loop(0, n_iters, body, 0)

        wait_out(tr_a, sem_oa)
        wait_out(tr_b, sem_ob)

        @pl.when(wid == 0)
        def _():
            pltpu.sync_copy(tail_hbm, in_a)
            transpose(in_a, st_a, n_i=EMB_DIM)
            compact(st_a, tr_a, n_i=EMB_DIM)
            pltpu.sync_copy(
                tr_a.at[pl.ds(0, BLK_WORDS // 2)],
                out_hbm.at[pl.ds(N_BLOCKS * BLK_WORDS, BLK_WORDS // 2)])

    return relayout


def _make_gather():
    mesh = plsc.VectorSubcoreMesh(core_axis_name="c", subcore_axis_name="s")

    @functools.partial(
        pl.kernel,
        out_type=jax.ShapeDtypeStruct((B_TOTAL, PAD_DIM), jnp.float32),
        mesh=mesh,
        scratch_types=[
            pltpu.VMEM((N_STREAMS, IDX_MINOR), jnp.int32),
            pltpu.VMEM((ROWS_PER_CHUNK, EMB_DIM), jnp.float32),
            pltpu.VMEM((ROWS_PER_CHUNK, EMB_DIM), jnp.float32),
            pltpu.SemaphoreType.DMA,
            pltpu.SemaphoreType.DMA,
        ],
        compiler_params=pltpu.CompilerParams(use_tc_tiling_on_sc=False),
    )
    def emb(x_hbm, table_hbm, out_hbm, idx_v, buf_a, buf_b, sem_a, sem_b):
        wid = lax.axis_index("s") * NC + lax.axis_index("c")
        base = wid * B_PER_W
        pltpu.sync_copy(x_hbm.at[wid], idx_v)

        def fire(c, buf, sem):
            for j in range(GROUP):
                pltpu.async_copy(
                    table_hbm.at[idx_v.at[c * GROUP + j]],
                    buf.at[pl.ds(j * IDX_MINOR, IDX_MINOR), :],
                    sem,
                )

        def drain(c, buf, sem):
            for j in range(GROUP):
                pltpu.make_async_copy(
                    table_hbm.at[idx_v.at[c * GROUP + j]],
                    buf.at[pl.ds(j * IDX_MINOR, IDX_MINOR), :],
                    sem,
                ).wait()

        def writeback(c, buf):
            pltpu.sync_copy(
                buf,
                out_hbm.at[pl.ds(base + c * ROWS_PER_CHUNK, ROWS_PER_CHUNK),
                           pl.ds(0, EMB_DIM)],
            )

        fire(0, buf_a, sem_a)

        def body(k, carry):
            c0 = 2 * k
            fire(c0 + 1, buf_b, sem_b)
            drain(c0, buf_a, sem_a)
            writeback(c0, buf_a)

            @pl.when(k < N_HALF - 1)
            def _():
                fire(c0 + 2, buf_a, sem_a)

            drain(c0 + 1, buf_b, sem_b)
            writeback(c0 + 1, buf_b)
            return carry

        lax.fori_loop(0, N_HALF, body, 0)

    return emb


_relayout = _make_relayout()
_emb = _make_gather()


@jax.jit
def kernel(x, table):
    x_r = x.reshape(NW, N_STREAMS, IDX_MINOR).astype(jnp.int32)
    tail_p = jnp.pad(table[N_BLOCKS * PAD_DIM:].T,
                     ((0, 0), (0, PAD_DIM - EMB_DIM)))
    table_lin = _relayout(table.T, tail_p).reshape(VOCAB, EMB_DIM)
    out_p = _emb(x_r, table_lin)
    return out_p[:, :EMB_DIM].reshape(BATCH, SEQ, EMB_DIM)


# speedup vs baseline: 1.3996x; 1.0946x over previous
import functools

import jax
import jax.numpy as jnp
from jax import lax
from jax.experimental import pallas as pl
from jax.experimental.pallas import tpu as pltpu
from jax.experimental.pallas import tpu_sc as plsc

BATCH = 4096
SEQ = 200
EMB_DIM = 64
PAD_DIM = 128
VOCAB = 1000000

NC = 2
NS = 16
NW = NC * NS
L = 16

N_BLOCKS = VOCAB // PAD_DIM
BLOCKS_PER_W = N_BLOCKS // NW + 1
BLK_WORDS = PAD_DIM * EMB_DIM
STRIDE = EMB_DIM + 1

B_TOTAL = BATCH * SEQ
B_PER_W = B_TOTAL // NW
IDX_MINOR = 128
N_STREAMS = B_PER_W // IDX_MINOR
GROUP = 4
ROWS_PER_CHUNK = GROUP * IDX_MINOR
N_CHUNKS = N_STREAMS // GROUP
N_HALF = N_CHUNKS // 2


def _make_relayout():
    mesh = plsc.VectorSubcoreMesh(core_axis_name="c", subcore_axis_name="s")

    @functools.partial(
        pl.kernel,
        out_type=jax.ShapeDtypeStruct((VOCAB * EMB_DIM,), jnp.float32),
        mesh=mesh,
        scratch_types=[
            pltpu.VMEM((EMB_DIM, PAD_DIM), jnp.float32),
            pltpu.VMEM((EMB_DIM, PAD_DIM), jnp.float32),
            pltpu.VMEM((PAD_DIM * STRIDE,), jnp.float32),
            pltpu.VMEM((PAD_DIM * STRIDE,), jnp.float32),
            pltpu.VMEM((BLK_WORDS,), jnp.float32),
            pltpu.VMEM((BLK_WORDS,), jnp.float32),
            pltpu.SemaphoreType.DMA,
            pltpu.SemaphoreType.DMA,
            pltpu.SemaphoreType.DMA,
            pltpu.SemaphoreType.DMA,
        ],
        compiler_params=pltpu.CompilerParams(
            use_tc_tiling_on_sc=True, needs_layout_passes=False),
    )
    def relayout(tt_hbm, tail_hbm, out_hbm, in_a, in_b, st_a, st_b,
                 tr_a, tr_b, sem_ia, sem_ib, sem_oa, sem_ob):
        wid = lax.axis_index("s") * NC + lax.axis_index("c")
        iota65 = lax.iota(jnp.int32, L) * STRIDE

        def blk(n):
            return n * NW + wid

        def fire_in(it, buf, sem):
            off = pl.multiple_of(it * PAD_DIM, PAD_DIM)
            pltpu.async_copy(
                tt_hbm.at[:, pl.ds(off, PAD_DIM)], buf, sem)

        def wait_in(buf, sem):
            pltpu.make_async_copy(
                tt_hbm.at[:, pl.ds(0, PAD_DIM)], buf, sem).wait()

        def transpose(buf, st, n_i=PAD_DIM):
            for d in range(EMB_DIM):
                for ib in range(n_i // L):
                    v = buf[d, pl.ds(ib * L, L)]
                    plsc.store_scatter(
                        st, [iota65 + (ib * L * STRIDE + d)], v)

        def compact(st, tr, n_i=PAD_DIM):
            for i in range(n_i):
                for k in range(EMB_DIM // L):
                    tr[pl.ds(i * EMB_DIM + k * L, L)] = (
                        st[pl.ds(i * STRIDE + k * L, L)])

        def fire_out(it, tr, sem):
            off = pl.multiple_of(it * BLK_WORDS, BLK_WORDS)
            pltpu.async_copy(
                tr, out_hbm.at[pl.ds(off, BLK_WORDS)], sem)

        def wait_out(tr, sem):
            pltpu.make_async_copy(
                tr, out_hbm.at[pl.ds(0, BLK_WORDS)], sem).wait()

        fire_in(blk(0), in_a, sem_ia)

        def body(k, carry):
            n0 = 2 * k

            @pl.when(blk(n0 + 1) < N_BLOCKS)
            def _():
                fire_in(blk(n0 + 1), in_b, sem_ib)

            @pl.when(blk(n0) < N_BLOCKS)
            def _():
                wait_in(in_a, sem_ia)

                @pl.when(k > 0)
                def _():
                    wait_out(tr_a, sem_oa)
                transpose(in_a, st_a)
                compact(st_a, tr_a)
                fire_out(blk(n0), tr_a, sem_oa)

            @pl.when(blk(n0 + 2) < N_BLOCKS)
            def _():
                fire_in(blk(n0 + 2), in_a, sem_ia)

            @pl.when(blk(n0 + 1) < N_BLOCKS)
            def _():
                wait_in(in_b, sem_ib)

                @pl.when(k > 0)
                def _():
                    wait_out(tr_b, sem_ob)
                transpose(in_b, st_b)
                compact(st_b, tr_b)
                fire_out(blk(n0 + 1), tr_b, sem_ob)
            return carry

        n_iters = (BLOCKS_PER_W + 1) // 2
        lax.fori_loop(0, n_iters, body, 0)

        wait_out(tr_a, sem_oa)
        wait_out(tr_b, sem_ob)

        @pl.when(wid == 0)
        def _():
            pltpu.sync_copy(tail_hbm, in_a)
            transpose(in_a, st_a, n_i=EMB_DIM)
            compact(st_a, tr_a, n_i=EMB_DIM)
            pltpu.sync_copy(
                tr_a.at[pl.ds(0, BLK_WORDS // 2)],
                out_hbm.at[pl.ds(N_BLOCKS * BLK_WORDS, BLK_WORDS // 2)])

    return relayout


def _make_gather():
    mesh = plsc.VectorSubcoreMesh(core_axis_name="c", subcore_axis_name="s")

    @functools.partial(
        pl.kernel,
        out_type=jax.ShapeDtypeStruct((B_TOTAL, PAD_DIM), jnp.float32),
        mesh=mesh,
        scratch_types=[
            pltpu.VMEM((N_STREAMS, IDX_MINOR), jnp.int32),
            pltpu.VMEM((ROWS_PER_CHUNK, EMB_DIM), jnp.float32),
            pltpu.VMEM((ROWS_PER_CHUNK, EMB_DIM), jnp.float32),
            pltpu.SemaphoreType.DMA,
            pltpu.SemaphoreType.DMA,
        ],
        compiler_params=pltpu.CompilerParams(use_tc_tiling_on_sc=False),
    )
    def emb(x_hbm, table_hbm, out_hbm, idx_v, buf_a, buf_b, sem_a, sem_b):
        wid = lax.axis_index("s") * NC + lax.axis_index("c")
        base = wid * B_PER_W
        pltpu.sync_copy(x_hbm.at[wid], idx_v)

        def fire(c, buf, sem):
            for j in range(GROUP):
                pltpu.async_copy(
                    table_hbm.at[idx_v.at[c * GROUP + j]],
                    buf.at[pl.ds(j * IDX_MINOR, IDX_MINOR), :],
                    sem,
                )

        def drain(c, buf, sem):
            for j in range(GROUP):
                pltpu.make_async_copy(
                    table_hbm.at[idx_v.at[c * GROUP + j]],
                    buf.at[pl.ds(j * IDX_MINOR, IDX_MINOR), :],
                    sem,
                ).wait()

        def writeback(c, buf):
            pltpu.sync_copy(
                buf,
                out_hbm.at[pl.ds(base + c * ROWS_PER_CHUNK, ROWS_PER_CHUNK),
                           pl.ds(0, EMB_DIM)],
            )

        fire(0, buf_a, sem_a)

        def body(k, carry):
            c0 = 2 * k
            fire(c0 + 1, buf_b, sem_b)
            drain(c0, buf_a, sem_a)
            writeback(c0, buf_a)

            @pl.when(k < N_HALF - 1)
            def _():
                fire(c0 + 2, buf_a, sem_a)

            drain(c0 + 1, buf_b, sem_b)
            writeback(c0 + 1, buf_b)
            return carry

        lax.fori_loop(0, N_HALF, body, 0)

    return emb


_relayout = _make_relayout()
_emb = _make_gather()


@jax.jit
def kernel(x, table):
    x_r = x.reshape(NW, N_STREAMS, IDX_MINOR).astype(jnp.int32)
    tail_p = jnp.pad(table[N_BLOCKS * PAD_DIM:].T,
                     ((0, 0), (0, PAD_DIM - EMB_DIM)))
    table_lin = _relayout(table.T, tail_p).reshape(VOCAB, EMB_DIM)
    out_p = _emb(x_r, table_lin)
    return out_p[:, :EMB_DIM].reshape(BATCH, SEQ, EMB_DIM)


# speedup vs baseline: 2.6838x; 1.9175x over previous
import functools

import jax
import jax.numpy as jnp
from jax import lax
from jax.experimental import pallas as pl
from jax.experimental.pallas import tpu as pltpu
from jax.experimental.pallas import tpu_sc as plsc

BATCH = 4096
SEQ = 200
EMB_DIM = 64
PAD_DIM = 128
VOCAB = 1000000

NC = 2
NS = 16
NW = NC * NS
L = 16

N_BLOCKS = VOCAB // PAD_DIM
BLOCKS_PER_W = N_BLOCKS // NW + 1
BLK_WORDS = PAD_DIM * EMB_DIM
STRIDE = EMB_DIM + 1

B_TOTAL = BATCH * SEQ
B_PER_W = B_TOTAL // NW
IDX_MINOR = 128
N_STREAMS = B_PER_W // IDX_MINOR
GROUP = 4
ROWS_PER_CHUNK = GROUP * IDX_MINOR
N_CHUNKS = N_STREAMS // GROUP
N_HALF = N_CHUNKS // 2


def _make_relayout():
    mesh = plsc.VectorSubcoreMesh(core_axis_name="c", subcore_axis_name="s")

    @functools.partial(
        pl.kernel,
        out_type=jax.ShapeDtypeStruct((VOCAB * EMB_DIM,), jnp.float32),
        mesh=mesh,
        scratch_types=[
            pltpu.VMEM((EMB_DIM, PAD_DIM), jnp.float32),
            pltpu.VMEM((EMB_DIM, PAD_DIM), jnp.float32),
            pltpu.VMEM((PAD_DIM * STRIDE,), jnp.float32),
            pltpu.VMEM((PAD_DIM * STRIDE,), jnp.float32),
            pltpu.VMEM((BLK_WORDS,), jnp.float32),
            pltpu.VMEM((BLK_WORDS,), jnp.float32),
            pltpu.SemaphoreType.DMA,
            pltpu.SemaphoreType.DMA,
            pltpu.SemaphoreType.DMA,
            pltpu.SemaphoreType.DMA,
        ],
        compiler_params=pltpu.CompilerParams(
            use_tc_tiling_on_sc=True, needs_layout_passes=False),
    )
    def relayout(tt_hbm, tail_hbm, out_hbm, in_a, in_b, st_a, st_b,
                 tr_a, tr_b, sem_ia, sem_ib, sem_oa, sem_ob):
        wid = lax.axis_index("s") * NC + lax.axis_index("c")
        iota65 = lax.iota(jnp.int32, L) * STRIDE

        def blk(n):
            return n * NW + wid

        def fire_in(it, buf, sem):
            off = pl.multiple_of(it * PAD_DIM, PAD_DIM)
            pltpu.async_copy(
                tt_hbm.at[:, pl.ds(off, PAD_DIM)], buf, sem)

        def wait_in(buf, sem):
            pltpu.make_async_copy(
                tt_hbm.at[:, pl.ds(0, PAD_DIM)], buf, sem).wait()

        def transpose(buf, st, n_i=PAD_DIM):
            @plsc.parallel_loop(0, EMB_DIM, unroll=8)
            def _(d):
                for ib in range(n_i // L):
                    v = buf[d, pl.ds(ib * L, L)]
                    plsc.store_scatter(
                        st, [iota65 + (ib * L * STRIDE + d)], v)

        def compact(st, tr, n_i=PAD_DIM):
            @plsc.parallel_loop(0, n_i, unroll=8)
            def _(i):
                for k in range(EMB_DIM // L):
                    tr[pl.ds(i * EMB_DIM + k * L, L)] = (
                        st[pl.ds(i * STRIDE + k * L, L)])

        def fire_out(it, tr, sem):
            off = pl.multiple_of(it * BLK_WORDS, BLK_WORDS)
            pltpu.async_copy(
                tr, out_hbm.at[pl.ds(off, BLK_WORDS)], sem)

        def wait_out(tr, sem):
            pltpu.make_async_copy(
                tr, out_hbm.at[pl.ds(0, BLK_WORDS)], sem).wait()

        fire_in(blk(0), in_a, sem_ia)

        def body(k, carry):
            n0 = 2 * k

            @pl.when(blk(n0 + 1) < N_BLOCKS)
            def _():
                fire_in(blk(n0 + 1), in_b, sem_ib)

            @pl.when(blk(n0) < N_BLOCKS)
            def _():
                wait_in(in_a, sem_ia)

                @pl.when(k > 0)
                def _():
                    wait_out(tr_a, sem_oa)
                transpose(in_a, st_a)
                compact(st_a, tr_a)
                fire_out(blk(n0), tr_a, sem_oa)

            @pl.when(blk(n0 + 2) < N_BLOCKS)
            def _():
                fire_in(blk(n0 + 2), in_a, sem_ia)

            @pl.when(blk(n0 + 1) < N_BLOCKS)
            def _():
                wait_in(in_b, sem_ib)

                @pl.when(k > 0)
                def _():
                    wait_out(tr_b, sem_ob)
                transpose(in_b, st_b)
                compact(st_b, tr_b)
                fire_out(blk(n0 + 1), tr_b, sem_ob)
            return carry

        n_iters = (BLOCKS_PER_W + 1) // 2
        lax.fori_loop(0, n_iters, body, 0)

        wait_out(tr_a, sem_oa)
        wait_out(tr_b, sem_ob)

        @pl.when(wid == 0)
        def _():
            pltpu.sync_copy(tail_hbm, in_a)
            transpose(in_a, st_a, n_i=EMB_DIM)
            compact(st_a, tr_a, n_i=EMB_DIM)
            pltpu.sync_copy(
                tr_a.at[pl.ds(0, BLK_WORDS // 2)],
                out_hbm.at[pl.ds(N_BLOCKS * BLK_WORDS, BLK_WORDS // 2)])

    return relayout


def _make_gather():
    mesh = plsc.VectorSubcoreMesh(core_axis_name="c", subcore_axis_name="s")

    @functools.partial(
        pl.kernel,
        out_type=jax.ShapeDtypeStruct((B_TOTAL, PAD_DIM), jnp.float32),
        mesh=mesh,
        scratch_types=[
            pltpu.VMEM((N_STREAMS, IDX_MINOR), jnp.int32),
            pltpu.VMEM((ROWS_PER_CHUNK, EMB_DIM), jnp.float32),
            pltpu.VMEM((ROWS_PER_CHUNK, EMB_DIM), jnp.float32),
            pltpu.SemaphoreType.DMA,
            pltpu.SemaphoreType.DMA,
        ],
        compiler_params=pltpu.CompilerParams(use_tc_tiling_on_sc=False),
    )
    def emb(x_hbm, table_hbm, out_hbm, idx_v, buf_a, buf_b, sem_a, sem_b):
        wid = lax.axis_index("s") * NC + lax.axis_index("c")
        base = wid * B_PER_W
        pltpu.sync_copy(x_hbm.at[wid], idx_v)

        def fire(c, buf, sem):
            for j in range(GROUP):
                pltpu.async_copy(
                    table_hbm.at[idx_v.at[c * GROUP + j]],
                    buf.at[pl.ds(j * IDX_MINOR, IDX_MINOR), :],
                    sem,
                )

        def drain(c, buf, sem):
            for j in range(GROUP):
                pltpu.make_async_copy(
                    table_hbm.at[idx_v.at[c * GROUP + j]],
                    buf.at[pl.ds(j * IDX_MINOR, IDX_MINOR), :],
                    sem,
                ).wait()

        def writeback(c, buf):
            pltpu.sync_copy(
                buf,
                out_hbm.at[pl.ds(base + c * ROWS_PER_CHUNK, ROWS_PER_CHUNK),
                           pl.ds(0, EMB_DIM)],
            )

        fire(0, buf_a, sem_a)

        def body(k, carry):
            c0 = 2 * k
            fire(c0 + 1, buf_b, sem_b)
            drain(c0, buf_a, sem_a)
            writeback(c0, buf_a)

            @pl.when(k < N_HALF - 1)
            def _():
                fire(c0 + 2, buf_a, sem_a)

            drain(c0 + 1, buf_b, sem_b)
            writeback(c0 + 1, buf_b)
            return carry

        lax.fori_loop(0, N_HALF, body, 0)

    return emb


_relayout = _make_relayout()
_emb = _make_gather()


@jax.jit
def kernel(x, table):
    x_r = x.reshape(NW, N_STREAMS, IDX_MINOR).astype(jnp.int32)
    tail_p = jnp.pad(table[N_BLOCKS * PAD_DIM:].T,
                     ((0, 0), (0, PAD_DIM - EMB_DIM)))
    table_lin = _relayout(table.T, tail_p).reshape(VOCAB, EMB_DIM)
    out_p = _emb(x_r, table_lin)
    return out_p[:, :EMB_DIM].reshape(BATCH, SEQ, EMB_DIM)


# speedup vs baseline: 3.4523x; 1.2864x over previous
import functools

import jax
import jax.numpy as jnp
from jax import lax
from jax.experimental import pallas as pl
from jax.experimental.pallas import tpu as pltpu
from jax.experimental.pallas import tpu_sc as plsc

BATCH = 4096
SEQ = 200
EMB_DIM = 64
PAD_DIM = 128
VOCAB = 1000000

NC = 2
NS = 16
NW = NC * NS
L = 16

N_BLOCKS = VOCAB // PAD_DIM
BLOCKS_PER_W = N_BLOCKS // NW + 1
BLK_WORDS = PAD_DIM * EMB_DIM
STRIDE = EMB_DIM + 1

B_TOTAL = BATCH * SEQ
B_PER_W = B_TOTAL // NW
IDX_MINOR = 128
N_STREAMS = B_PER_W // IDX_MINOR
GROUP = 4
ROWS_PER_CHUNK = GROUP * IDX_MINOR
N_CHUNKS = N_STREAMS // GROUP
N_HALF = N_CHUNKS // 2


def _make_relayout():
    mesh = plsc.VectorSubcoreMesh(core_axis_name="c", subcore_axis_name="s")

    @functools.partial(
        pl.kernel,
        out_type=jax.ShapeDtypeStruct((VOCAB * EMB_DIM,), jnp.float32),
        mesh=mesh,
        scratch_types=[
            pltpu.VMEM((EMB_DIM, PAD_DIM), jnp.float32),
            pltpu.VMEM((EMB_DIM, PAD_DIM), jnp.float32),
            pltpu.VMEM((PAD_DIM * STRIDE,), jnp.float32),
            pltpu.VMEM((PAD_DIM * STRIDE,), jnp.float32),
            pltpu.VMEM((BLK_WORDS,), jnp.float32),
            pltpu.VMEM((BLK_WORDS,), jnp.float32),
            pltpu.SemaphoreType.DMA,
            pltpu.SemaphoreType.DMA,
            pltpu.SemaphoreType.DMA,
            pltpu.SemaphoreType.DMA,
        ],
        compiler_params=pltpu.CompilerParams(
            use_tc_tiling_on_sc=True, needs_layout_passes=False),
    )
    def relayout(tt_hbm, tail_hbm, out_hbm, in_a, in_b, st_a, st_b,
                 tr_a, tr_b, sem_ia, sem_ib, sem_oa, sem_ob):
        wid = lax.axis_index("s") * NC + lax.axis_index("c")
        iota65 = lax.iota(jnp.int32, L) * STRIDE

        def blk(n):
            return n * NW + wid

        def fire_in(it, buf, sem):
            off = pl.multiple_of(it * PAD_DIM, PAD_DIM)
            pltpu.async_copy(
                tt_hbm.at[:, pl.ds(off, PAD_DIM)], buf, sem)

        def wait_in(buf, sem):
            pltpu.make_async_copy(
                tt_hbm.at[:, pl.ds(0, PAD_DIM)], buf, sem).wait()

        def transpose(buf, st, n_i=PAD_DIM):
            @plsc.parallel_loop(0, EMB_DIM, unroll=8)
            def _(d):
                for ib in range(n_i // L):
                    v = buf[d, pl.ds(ib * L, L)]
                    plsc.store_scatter(
                        st, [iota65 + (ib * L * STRIDE + d)], v)

        def compact(st, tr, n_i=PAD_DIM):
            @plsc.parallel_loop(0, n_i, unroll=8)
            def _(i):
                for k in range(EMB_DIM // L):
                    tr[pl.ds(i * EMB_DIM + k * L, L)] = (
                        st[pl.ds(i * STRIDE + k * L, L)])

        def fire_out(it, tr, sem):
            off = pl.multiple_of(it * BLK_WORDS, BLK_WORDS)
            pltpu.async_copy(
                tr, out_hbm.at[pl.ds(off, BLK_WORDS)], sem)

        def wait_out(tr, sem):
            pltpu.make_async_copy(
                tr, out_hbm.at[pl.ds(0, BLK_WORDS)], sem).wait()

        fire_in(blk(0), in_a, sem_ia)

        def body(k, carry):
            n0 = 2 * k

            @pl.when(blk(n0 + 1) < N_BLOCKS)
            def _():
                fire_in(blk(n0 + 1), in_b, sem_ib)

            @pl.when(blk(n0) < N_BLOCKS)
            def _():
                wait_in(in_a, sem_ia)

                @pl.when(k > 0)
                def _():
                    wait_out(tr_a, sem_oa)
                transpose(in_a, st_a)
                compact(st_a, tr_a)
                fire_out(blk(n0), tr_a, sem_oa)

            @pl.when(blk(n0 + 2) < N_BLOCKS)
            def _():
                fire_in(blk(n0 + 2), in_a, sem_ia)

            @pl.when(blk(n0 + 1) < N_BLOCKS)
            def _():
                wait_in(in_b, sem_ib)

                @pl.when(k > 0)
                def _():
                    wait_out(tr_b, sem_ob)
                transpose(in_b, st_b)
                compact(st_b, tr_b)
                fire_out(blk(n0 + 1), tr_b, sem_ob)
            return carry

        n_iters = (BLOCKS_PER_W + 1) // 2
        lax.fori_loop(0, n_iters, body, 0)

        wait_out(tr_a, sem_oa)
        wait_out(tr_b, sem_ob)

        @pl.when(wid == 0)
        def _():
            pltpu.sync_copy(tail_hbm, in_a)
            transpose(in_a, st_a, n_i=EMB_DIM)
            compact(st_a, tr_a, n_i=EMB_DIM)
            pltpu.sync_copy(
                tr_a.at[pl.ds(0, BLK_WORDS // 2)],
                out_hbm.at[pl.ds(N_BLOCKS * BLK_WORDS, BLK_WORDS // 2)])

    return relayout


N_GBLOCKS = SEQ * (BATCH // PAD_DIM)
GBLK_PER_W = N_GBLOCKS // NW
DT = EMB_DIM // 8
ST2_WORDS = EMB_DIM * (PAD_DIM + 1)
OBLK_WORDS = PAD_DIM * EMB_DIM


def _make_gather():
    mesh = plsc.VectorSubcoreMesh(core_axis_name="c", subcore_axis_name="s")

    @functools.partial(
        pl.kernel,
        out_type=jax.ShapeDtypeStruct((SEQ * DT * 32 * 8 * PAD_DIM,),
                                      jnp.float32),
        mesh=mesh,
        scratch_types=[
            pltpu.VMEM((GBLK_PER_W, IDX_MINOR), jnp.int32),
            pltpu.VMEM((IDX_MINOR, EMB_DIM), jnp.float32),
            pltpu.VMEM((IDX_MINOR, EMB_DIM), jnp.float32),
            pltpu.VMEM((ST2_WORDS,), jnp.float32),
            pltpu.VMEM((OBLK_WORDS,), jnp.float32),
            pltpu.VMEM((OBLK_WORDS,), jnp.float32),
            pltpu.SemaphoreType.DMA,
            pltpu.SemaphoreType.DMA,
            pltpu.SemaphoreType.DMA,
            pltpu.SemaphoreType.DMA,
        ],
        compiler_params=pltpu.CompilerParams(
            use_tc_tiling_on_sc=False, needs_layout_passes=False),
    )
    def emb(x_hbm, table_hbm, out_hbm, idx_v, buf_a, buf_b, st2, trc_a,
            trc_b, sem_a, sem_b, sem_oa, sem_ob):
        wid = lax.axis_index("s") * NC + lax.axis_index("c")
        g0 = wid * GBLK_PER_W
        jvec = lax.iota(jnp.int32, L) * (PAD_DIM + 1)
        pltpu.sync_copy(x_hbm.at[pl.ds(g0, GBLK_PER_W), :], idx_v)

        def fire(c, buf, sem):
            pltpu.async_copy(table_hbm.at[idx_v.at[c]], buf, sem)

        def drain(c, buf, sem):
            pltpu.make_async_copy(
                table_hbm.at[idx_v.at[c]], buf, sem).wait()

        def transpose(buf):
            @plsc.parallel_loop(0, IDX_MINOR, unroll=8)
            def _(bb):
                for k in range(EMB_DIM // L):
                    v = buf[bb, pl.ds(k * L, L)]
                    plsc.store_scatter(
                        st2, [jvec + (k * L * (PAD_DIM + 1) + bb)], v)

        def compact(trc):
            @plsc.parallel_loop(0, EMB_DIM, unroll=8)
            def _(m):
                for k in range(PAD_DIM // L):
                    trc[pl.ds(m * PAD_DIM + k * L, L)] = (
                        st2[pl.ds(m * (PAD_DIM + 1) + k * L, L)])

        def fire_out(c, trc, sem):
            s = c // 32
            bt = c - s * 32
            base = s * (DT * 32 * 8 * PAD_DIM) + bt * (8 * PAD_DIM)
            for dt in range(DT):
                off = pl.multiple_of(base + dt * (32 * 8 * PAD_DIM),
                                     8 * PAD_DIM)
                pltpu.async_copy(
                    trc.at[pl.ds(dt * 8 * PAD_DIM, 8 * PAD_DIM)],
                    out_hbm.at[pl.ds(off, 8 * PAD_DIM)], sem)

        def wait_out(trc, sem):
            for dt in range(DT):
                pltpu.make_async_copy(
                    trc.at[pl.ds(dt * 8 * PAD_DIM, 8 * PAD_DIM)],
                    out_hbm.at[pl.ds(0, 8 * PAD_DIM)], sem).wait()

        fire(0, buf_a, sem_a)

        def body(k, carry):
            c0 = 2 * k
            fire(c0 + 1, buf_b, sem_b)
            drain(c0, buf_a, sem_a)
            transpose(buf_a)

            @pl.when(k > 0)
            def _():
                wait_out(trc_a, sem_oa)
            compact(trc_a)
            fire_out(g0 + c0, trc_a, sem_oa)

            @pl.when(k < GBLK_PER_W // 2 - 1)
            def _():
                fire(c0 + 2, buf_a, sem_a)

            drain(c0 + 1, buf_b, sem_b)
            transpose(buf_b)

            @pl.when(k > 0)
            def _():
                wait_out(trc_b, sem_ob)
            compact(trc_b)
            fire_out(g0 + c0 + 1, trc_b, sem_ob)
            return carry

        lax.fori_loop(0, GBLK_PER_W // 2, body, 0)
        wait_out(trc_a, sem_oa)
        wait_out(trc_b, sem_ob)

    return emb


_relayout = _make_relayout()
_emb = _make_gather()


@jax.jit
def kernel(x, table):
    xt_r = x.T.reshape(N_GBLOCKS, IDX_MINOR).astype(jnp.int32)
    tail_p = jnp.pad(table[N_BLOCKS * PAD_DIM:].T,
                     ((0, 0), (0, PAD_DIM - EMB_DIM)))
    table_lin = _relayout(table.T, tail_p).reshape(VOCAB, EMB_DIM)
    o5 = _emb(xt_r, table_lin).reshape(SEQ, DT, 32, 8, PAD_DIM)
    return o5.transpose(2, 4, 0, 1, 3).reshape(BATCH, SEQ, EMB_DIM)
